# baseline (speedup 1.0000x reference)
import functools

import jax
import jax.numpy as jnp
from jax import lax
from jax.experimental import pallas as pl
from jax.experimental.pallas import tpu as pltpu
from jax.experimental.pallas import tpu_sc as plsc

E = 64
K = 2
D = 768
F = 1536
T = 128
L = 16
NCORE = 2
NSUB = 16
TG = T // L


def _logits_body(x_ref, Wg_ref, bg_ref, out_ref):
    out_ref[:] = lax.dot_general(
        Wg_ref[:], x_ref[:], (((0,), (1,)), ((), ())),
        preferred_element_type=jnp.float32) + bg_ref[:]


def _route_body(lt_hbm, w_hbm, ltv, exv, pv, wout, sem):
    cid = lax.axis_index("c")
    sid = lax.axis_index("s")
    q = sid % 4
    tg = cid * 4 + sid // 4

    @pl.when(q == 0)
    def _route():
        pltpu.sync_copy(lt_hbm, ltv)
        off = tg * L
        mv = ltv[pl.ds(off, L)]
        for ge in range(1, E):
            mv = jnp.maximum(mv, ltv[pl.ds(ge * T + off, L)])
        s = jnp.zeros((L,), jnp.float32)
        for ge in range(E):
            ex = jnp.exp(ltv[pl.ds(ge * T + off, L)] - mv)
            exv[pl.ds(ge * L, L)] = ex
            s = s + ex
        m1 = jnp.full((L,), -1.0, jnp.float32)
        for ge in range(E):
            p = exv[pl.ds(ge * L, L)] / s
            pv[pl.ds(ge * L, L)] = p
            m1 = jnp.maximum(m1, p)
        big = jnp.full((L,), E, jnp.int32)
        i1 = big
        for ge in range(E):
            p = pv[pl.ds(ge * L, L)]
            i1 = jnp.minimum(i1, jnp.where(p == m1, ge, E))
        m2 = jnp.full((L,), -1.0, jnp.float32)
        for ge in range(E):
            p = pv[pl.ds(ge * L, L)]
            m2 = jnp.maximum(m2, jnp.where(i1 == ge, -1.0, p))
        i2 = big
        for ge in range(E):
            p = jnp.where(i1 == ge, -1.0, pv[pl.ds(ge * L, L)])
            i2 = jnp.minimum(i2, jnp.where(p == m2, ge, E))
        denom = m1 + m2
        a1 = m1 / denom
        a2 = m2 / denom
        for ge in range(E):
            wge = (jnp.where(i1 == ge, a1, 0.0)
                   + jnp.where(i2 == ge, a2, 0.0))
            wout[pl.ds(ge * L, L)] = wge
        descs = [pltpu.make_async_copy(wout.at[pl.ds(ge * L, L)],
                                       w_hbm.at[pl.ds(ge * T + off, L)],
                                       sem)
                 for ge in range(E)]
        for dsc in descs:
            dsc.start()
        for dsc in descs:
            dsc.wait()


def _route_sc(lt):
    mesh = plsc.VectorSubcoreMesh(core_axis_name="c", subcore_axis_name="s",
                                  num_cores=NCORE, num_subcores=NSUB)
    fn = functools.partial(
        pl.kernel, mesh=mesh,
        out_type=jax.ShapeDtypeStruct((E * T,), jnp.float32),
        scratch_types=[
            pltpu.VMEM((E * T,), jnp.float32),
            pltpu.VMEM((E * L,), jnp.float32),
            pltpu.VMEM((E * L,), jnp.float32),
            pltpu.VMEM((E * L,), jnp.float32),
            pltpu.SemaphoreType.DMA,
        ],
    )(_route_body)
    return fn(lt)


NPRE = 8


def _pre_body(x_ref, W1_ref, b1_ref, W2_ref, o_ref):
    xb = x_ref[:]
    h = jnp.maximum(
        jnp.dot(xb, W1_ref[0], preferred_element_type=jnp.float32)
        + b1_ref[0, 0, :], 0.0)
    o_ref[0] = jnp.dot(h, W2_ref[0], preferred_element_type=jnp.float32)


def _moe_body(x_ref, wT_ref, W1_ref, b1_ref, W2_ref, b2_ref, o_ref,
              out_ref, acc_ref):
    e = pl.program_id(0)

    @pl.when(e == 0)
    def _init():
        acc_ref[:] = jnp.zeros_like(acc_ref)

    ridx = jax.lax.broadcasted_iota(jnp.int32, (E, 1), 0)
    onehot = (ridx == e).astype(jnp.float32)
    wcol = lax.dot_general(wT_ref[:], onehot, (((0,), (0,)), ((), ())),
                           preferred_element_type=jnp.float32)

    @pl.when(e < NPRE)
    def _buffered():
        acc_ref[:] += wcol * o_ref[0]

    @pl.when(e >= NPRE)
    def _stream():
        xb = x_ref[:]
        h = jnp.maximum(
            jnp.dot(xb, W1_ref[0], preferred_element_type=jnp.float32)
            + b1_ref[0, 0, :], 0.0)
        o = jnp.dot(h, W2_ref[0], preferred_element_type=jnp.float32)
        acc_ref[:] += wcol * o

    @pl.when(e == E - 1)
    def _finish():
        out_ref[:] = acc_ref[:] + lax.dot_general(
            wT_ref[:], b2_ref[:], (((0,), (0,)), ((), ())),
            preferred_element_type=jnp.float32)


def kernel(x, Wg, bg, W1, b1, W2, b2):
    B, S, _ = x.shape
    xf = x.reshape(T, D)
    b1r = b1.reshape(E, 1, F)
    lt = pl.pallas_call(
        _logits_body,
        in_specs=[
            pl.BlockSpec((T, D), lambda: (0, 0)),
            pl.BlockSpec((D, E), lambda: (0, 0)),
            pl.BlockSpec((E, 1), lambda: (0, 0)),
        ],
        out_specs=pl.BlockSpec((E, T), lambda: (0, 0)),
        out_shape=jax.ShapeDtypeStruct((E, T), jnp.float32),
    )(xf, Wg, bg.reshape(E, 1))
    wT = _route_sc(lt.reshape(-1)).reshape(E, T)
    obuf = pl.pallas_call(
        _pre_body,
        grid=(NPRE,),
        in_specs=[
            pl.BlockSpec((T, D), lambda e: (0, 0)),
            pl.BlockSpec((1, D, F), lambda e: (e, 0, 0)),
            pl.BlockSpec((1, 1, F), lambda e: (e, 0, 0)),
            pl.BlockSpec((1, F, D), lambda e: (e, 0, 0)),
        ],
        out_specs=pl.BlockSpec((1, T, D), lambda e: (e, 0, 0)),
        out_shape=jax.ShapeDtypeStruct((NPRE, T, D), jnp.float32),
    )(xf, W1, b1r, W2)
    out = pl.pallas_call(
        _moe_body,
        grid=(E,),
        in_specs=[
            pl.BlockSpec((T, D), lambda e: (0, 0)),
            pl.BlockSpec((E, T), lambda e: (0, 0)),
            pl.BlockSpec((1, D, F), lambda e: (jnp.maximum(e, NPRE), 0, 0)),
            pl.BlockSpec((1, 1, F), lambda e: (jnp.maximum(e, NPRE), 0, 0)),
            pl.BlockSpec((1, F, D), lambda e: (jnp.maximum(e, NPRE), 0, 0)),
            pl.BlockSpec((E, D), lambda e: (0, 0)),
            pl.BlockSpec((1, T, D), lambda e: (jnp.minimum(e, NPRE - 1), 0, 0)),
        ],
        out_specs=pl.BlockSpec((T, D), lambda e: (0, 0)),
        out_shape=jax.ShapeDtypeStruct((T, D), jnp.float32),
        scratch_shapes=[
            pltpu.VMEM((T, D), jnp.float32),
        ],
    )(xf, wT, W1, b1r, W2, b2, obuf)
    return out.reshape(B, S, D)


# speedup vs baseline: 1.0333x; 1.0333x over previous
import functools

import jax
import jax.numpy as jnp
from jax import lax
from jax.experimental import pallas as pl
from jax.experimental.pallas import tpu as pltpu
from jax.experimental.pallas import tpu_sc as plsc

E = 64
K = 2
D = 768
F = 1536
T = 128
L = 16
NCORE = 2
NSUB = 16
TG = T // L


def _logits_body(x_ref, Wg_ref, bg_ref, out_ref):
    out_ref[:] = lax.dot_general(
        Wg_ref[:], x_ref[:], (((0,), (1,)), ((), ())),
        preferred_element_type=jnp.float32) + bg_ref[:]


def _route_body(lt_hbm, w_hbm, ltv, exv, pv, wout, sem):
    cid = lax.axis_index("c")
    sid = lax.axis_index("s")
    q = sid % 4
    tg = cid * 4 + sid // 4

    @pl.when(q == 0)
    def _route():
        pltpu.sync_copy(lt_hbm, ltv)
        off = tg * L
        mv = ltv[pl.ds(off, L)]
        for ge in range(1, E):
            mv = jnp.maximum(mv, ltv[pl.ds(ge * T + off, L)])
        s = jnp.zeros((L,), jnp.float32)
        for ge in range(E):
            ex = jnp.exp(ltv[pl.ds(ge * T + off, L)] - mv)
            exv[pl.ds(ge * L, L)] = ex
            s = s + ex
        m1 = jnp.full((L,), -1.0, jnp.float32)
        for ge in range(E):
            p = exv[pl.ds(ge * L, L)] / s
            pv[pl.ds(ge * L, L)] = p
            m1 = jnp.maximum(m1, p)
        big = jnp.full((L,), E, jnp.int32)
        i1 = big
        for ge in range(E):
            p = pv[pl.ds(ge * L, L)]
            i1 = jnp.minimum(i1, jnp.where(p == m1, ge, E))
        m2 = jnp.full((L,), -1.0, jnp.float32)
        for ge in range(E):
            p = pv[pl.ds(ge * L, L)]
            m2 = jnp.maximum(m2, jnp.where(i1 == ge, -1.0, p))
        i2 = big
        for ge in range(E):
            p = jnp.where(i1 == ge, -1.0, pv[pl.ds(ge * L, L)])
            i2 = jnp.minimum(i2, jnp.where(p == m2, ge, E))
        denom = m1 + m2
        a1 = m1 / denom
        a2 = m2 / denom
        for ge in range(E):
            wge = (jnp.where(i1 == ge, a1, 0.0)
                   + jnp.where(i2 == ge, a2, 0.0))
            wout[pl.ds(ge * L, L)] = wge
        descs = [pltpu.make_async_copy(wout.at[pl.ds(ge * L, L)],
                                       w_hbm.at[pl.ds(ge * T + off, L)],
                                       sem)
                 for ge in range(E)]
        for dsc in descs:
            dsc.start()
        for dsc in descs:
            dsc.wait()


def _route_sc(lt):
    mesh = plsc.VectorSubcoreMesh(core_axis_name="c", subcore_axis_name="s",
                                  num_cores=NCORE, num_subcores=NSUB)
    fn = functools.partial(
        pl.kernel, mesh=mesh,
        out_type=jax.ShapeDtypeStruct((E * T,), jnp.float32),
        scratch_types=[
            pltpu.VMEM((E * T,), jnp.float32),
            pltpu.VMEM((E * L,), jnp.float32),
            pltpu.VMEM((E * L,), jnp.float32),
            pltpu.VMEM((E * L,), jnp.float32),
            pltpu.SemaphoreType.DMA,
        ],
    )(_route_body)
    return fn(lt)


def _moe_body(x_ref, wT_ref, W1_ref, b1_ref, W2_ref, b2_ref,
              out_ref, acc_ref):
    e = pl.program_id(0)

    @pl.when(e == 0)
    def _init():
        acc_ref[:] = jnp.zeros_like(acc_ref)

    xb = x_ref[:]
    h = jnp.maximum(
        jnp.dot(xb, W1_ref[0], preferred_element_type=jnp.float32)
        + b1_ref[0, 0, :], 0.0)
    o = jnp.dot(h, W2_ref[0], preferred_element_type=jnp.float32)
    ridx = jax.lax.broadcasted_iota(jnp.int32, (E, 1), 0)
    onehot = (ridx == e).astype(jnp.float32)
    wcol = lax.dot_general(wT_ref[:], onehot, (((0,), (0,)), ((), ())),
                           preferred_element_type=jnp.float32)
    acc_ref[:] += wcol * o

    @pl.when(e == E - 1)
    def _finish():
        out_ref[:] = acc_ref[:] + lax.dot_general(
            wT_ref[:], b2_ref[:], (((0,), (0,)), ((), ())),
            preferred_element_type=jnp.float32)


def kernel(x, Wg, bg, W1, b1, W2, b2):
    B, S, _ = x.shape
    xf = x.reshape(T, D)
    b1r = b1.reshape(E, 1, F)
    lt = pl.pallas_call(
        _logits_body,
        in_specs=[
            pl.BlockSpec((T, D), lambda: (0, 0)),
            pl.BlockSpec((D, E), lambda: (0, 0)),
            pl.BlockSpec((E, 1), lambda: (0, 0)),
        ],
        out_specs=pl.BlockSpec((E, T), lambda: (0, 0)),
        out_shape=jax.ShapeDtypeStruct((E, T), jnp.float32),
    )(xf, Wg, bg.reshape(E, 1))
    wT = _route_sc(lt.reshape(-1)).reshape(E, T)
    out = pl.pallas_call(
        _moe_body,
        grid=(E,),
        in_specs=[
            pl.BlockSpec((T, D), lambda e: (0, 0)),
            pl.BlockSpec((E, T), lambda e: (0, 0)),
            pl.BlockSpec((1, D, F), lambda e: (e, 0, 0)),
            pl.BlockSpec((1, 1, F), lambda e: (e, 0, 0)),
            pl.BlockSpec((1, F, D), lambda e: (e, 0, 0)),
            pl.BlockSpec((E, D), lambda e: (0, 0)),
        ],
        out_specs=pl.BlockSpec((T, D), lambda e: (0, 0)),
        out_shape=jax.ShapeDtypeStruct((T, D), jnp.float32),
        scratch_shapes=[
            pltpu.VMEM((T, D), jnp.float32),
        ],
    )(xf, wT, W1, b1r, W2, b2)
    return out.reshape(B, S, D)
